# baseline (speedup 1.0000x reference)
import functools

import jax
import jax.numpy as jnp
from jax import lax
from jax.experimental import pallas as pl
from jax.experimental.pallas import tpu as pltpu
from jax.experimental.pallas import tpu_sc as plsc

N = 10000
NPAD = 10240
E = 320000
D = 128
NC = 2
NS = 16
NW = NC * NS
C = 64
NCHUNK = E // C
KMAX = -(-NCHUNK // NW)
RPT = NPAD // NS
BN = 400
GRID = N // BN


def _pack_logits(a_s, a_d):
    su = lax.bitcast_convert_type(a_s.astype(jnp.bfloat16), jnp.uint16)
    du = lax.bitcast_convert_type(a_d.astype(jnp.bfloat16), jnp.uint16)
    return (du.astype(jnp.int32) << 16) | su.astype(jnp.int32)


def _tc_in_body(x_ref, w_ref, as_ref, ad_ref, h_ref, pk_ref):
    h = jnp.dot(x_ref[...], w_ref[...], preferred_element_type=jnp.float32)
    h_ref[...] = h
    a_s = jnp.dot(h, as_ref[...], preferred_element_type=jnp.float32)
    a_d = jnp.dot(h, ad_ref[...], preferred_element_type=jnp.float32)
    pk_ref[...] = _pack_logits(a_s, a_d)


def _tc_mid_body(np_ref, dp_ref, b1_ref, w_ref, as_ref, ad_ref,
                 h_ref, pk_ref):
    num = np_ref[0] + np_ref[1]
    den = dp_ref[0] + dp_ref[1]
    h1 = jnp.maximum(num / (den + 1e-16) + b1_ref[...], 0.0)
    h2 = jnp.dot(h1, w_ref[...], preferred_element_type=jnp.float32)
    h_ref[...] = h2
    a_s = jnp.dot(h2, as_ref[...], preferred_element_type=jnp.float32)
    a_d = jnp.dot(h2, ad_ref[...], preferred_element_type=jnp.float32)
    pk_ref[...] = _pack_logits(a_s, a_d)


def _tc_out_body(np_ref, dp_ref, b2_ref, out_ref):
    num = np_ref[0] + np_ref[1]
    den = dp_ref[0] + dp_ref[1]
    out_ref[...] = num / (den + 1e-16) + b2_ref[...]


_tc_in = pl.pallas_call(
    _tc_in_body,
    grid=(GRID,),
    in_specs=[
        pl.BlockSpec((BN, D), lambda i: (i, 0)),
        pl.BlockSpec((D, D), lambda i: (0, 0)),
        pl.BlockSpec((D, 1), lambda i: (0, 0)),
        pl.BlockSpec((D, 1), lambda i: (0, 0)),
    ],
    out_specs=[
        pl.BlockSpec((BN, D), lambda i: (i, 0)),
        pl.BlockSpec((BN, 1), lambda i: (i, 0)),
    ],
    out_shape=[
        jax.ShapeDtypeStruct((N, D), jnp.float32),
        jax.ShapeDtypeStruct((N, 1), jnp.int32),
    ],
)

_tc_mid = pl.pallas_call(
    _tc_mid_body,
    grid=(GRID,),
    in_specs=[
        pl.BlockSpec((2, BN, D), lambda i: (0, i, 0)),
        pl.BlockSpec((2, BN, 1), lambda i: (0, i, 0)),
        pl.BlockSpec((1, D), lambda i: (0, 0)),
        pl.BlockSpec((D, D), lambda i: (0, 0)),
        pl.BlockSpec((D, 1), lambda i: (0, 0)),
        pl.BlockSpec((D, 1), lambda i: (0, 0)),
    ],
    out_specs=[
        pl.BlockSpec((BN, D), lambda i: (i, 0)),
        pl.BlockSpec((BN, 1), lambda i: (i, 0)),
    ],
    out_shape=[
        jax.ShapeDtypeStruct((N, D), jnp.float32),
        jax.ShapeDtypeStruct((N, 1), jnp.int32),
    ],
)

_tc_out = pl.pallas_call(
    _tc_out_body,
    grid=(GRID,),
    in_specs=[
        pl.BlockSpec((2, BN, D), lambda i: (0, i, 0)),
        pl.BlockSpec((2, BN, 1), lambda i: (0, i, 0)),
        pl.BlockSpec((1, D), lambda i: (0, 0)),
    ],
    out_specs=pl.BlockSpec((BN, D), lambda i: (i, 0)),
    out_shape=jax.ShapeDtypeStruct((N, D), jnp.float32),
)


def _sc_body(h_hbm, pk_hbm, src_hbm, dst_hbm,
             num_hbm, den_hbm, hbf_hbm,
             acc, accd,
             pk_t,
             src0, src1, src2, dst0, dst1, dst2, w0, w1, rows0, rows1,
             srows0, srows1, zd_v,
             g0, g1, s0, s1, i0, i1, i2):
    cid = lax.axis_index("c")
    sid = lax.axis_index("s")
    wkr = sid * NC + cid

    srcs = (src0, src1, src2)
    dsts = (dst0, dst1, dst2)
    ws = (w0, w1)
    rows = (rows0, rows1)
    srows = (srows0, srows1)
    gsem = (g0, g1)
    ssem = (s0, s1)
    isem = (i0, i1, i2)

    pltpu.sync_copy(pk_hbm, pk_t)

    zeros16 = jnp.zeros((16,), jnp.float32)

    def _zero_row(i, _):
        for d in range(D // 16):
            srows0[i, pl.ds(16 * d, 16)] = zeros16
        return 0

    lax.fori_loop(0, C, _zero_row, 0)

    def _zero_zd(i, _):
        zd_v[pl.ds(16 * i, 16)] = zeros16
        return 0

    lax.fori_loop(0, RPT // 16, _zero_zd, 0)

    for cpy in range(RPT // C):
        pltpu.sync_copy(srows0, acc.at[pl.ds(sid * RPT + cpy * C, C)])
    pltpu.sync_copy(zd_v, accd.at[pl.ds(sid * RPT, RPT)])

    PKR = 40
    def _pack_chunk(c, _):
        @pl.when(c < N // PKR)
        def _():
            base = c * PKR
            pltpu.sync_copy(h_hbm.at[pl.ds(base, PKR)], srows1.at[pl.ds(0, PKR)])
            for i in range(PKR):
                for g in range(D // 32):
                    a = srows1[i, pl.ds(32 * g, 16)]
                    b = srows1[i, pl.ds(32 * g + 16, 16)]
                    rows1[i, pl.ds(32 * g, 32)] = plsc.pack(
                        a, b, format=plsc.PackFormat.INTERLEAVED)
            pltpu.sync_copy(rows1.at[pl.ds(0, PKR)],
                            hbf_hbm.at[pl.ds(base, PKR)])
        return 0

    lax.fori_loop(0, -(-(N // PKR) // NS), lambda t, _: _pack_chunk(t * NS + sid, 0), 0)
    plsc.subcore_barrier()

    def _compute_w(p, r):
        himask = jnp.full((16,), -65536, jnp.int32)
        for j in range(C // 16):
            si = srcs[r][pl.ds(16 * j, 16)]
            di = dsts[r][pl.ds(16 * j, 16)]
            vs = plsc.load_gather(pk_t, [si])
            vd = plsc.load_gather(pk_t, [di])
            a_s = plsc.bitcast(vs << 16, jnp.float32)
            a_d = plsc.bitcast(vd & himask, jnp.float32)
            z = a_s + a_d
            e = jnp.where(z >= 0.0, z, z * 0.2)
            ws[p][pl.ds(16 * j, 16)] = jnp.exp(e)

    def _scale(p):
        def body(j, _):
            wv = ws[p][pl.ds(16 * j, 16)]
            for e in range(16):
                wb = jnp.full((16,), wv[e], jnp.float32)
                row = 16 * j + e
                for g in range(D // 32):
                    v = rows[p][row, pl.ds(32 * g, 32)]
                    a, b = plsc.unpack(v, format=plsc.PackFormat.INTERLEAVED)
                    srows[p][row, pl.ds(32 * g, 16)] = a * wb
                    srows[p][row, pl.ds(32 * g + 16, 16)] = b * wb
            return 0

        lax.fori_loop(0, C // 16, body, 0)

    def _fetch_idx(k, r):
        base = (k * NW + wkr) * C
        pltpu.async_copy(src_hbm.at[pl.ds(base, C)], srcs[r], isem[r])
        pltpu.async_copy(dst_hbm.at[pl.ds(base, C)], dsts[r], isem[r])

    def _drain_idx(r):
        pltpu.make_async_copy(src_hbm.at[pl.ds(0, C)], srcs[r], isem[r]).wait()
        pltpu.make_async_copy(dst_hbm.at[pl.ds(0, C)], dsts[r], isem[r]).wait()

    _fetch_idx(0, 0)
    _fetch_idx(1, 1)
    _drain_idx(0)
    pltpu.async_copy(hbf_hbm.at[srcs[0]], rows[0], gsem[0])

    def _phase(k, p, r):
        q = 1 - p
        r1 = (r + 1) % 3
        r2 = (r + 2) % 3
        ck = k * NW + wkr
        valid = ck < NCHUNK
        prev_valid = jnp.logical_and(k >= 1, (k - 1) * NW + wkr < NCHUNK)
        next_valid = (k + 1) * NW + wkr < NCHUNK
        next2_valid = (k + 2) * NW + wkr < NCHUNK

        @pl.when(prev_valid)
        def _():
            pltpu.make_async_copy(srows[q], acc.at[dsts[r2]], ssem[q]).wait()
            pltpu.make_async_copy(ws[q], accd.at[dsts[r2]], ssem[q]).wait()

        @pl.when(next2_valid)
        def _():
            _fetch_idx(k + 2, r2)

        @pl.when(next_valid)
        def _():
            _drain_idx(r1)
            pltpu.async_copy(hbf_hbm.at[srcs[r1]], rows[q], gsem[q])

        @pl.when(valid)
        def _():
            _compute_w(p, r)
            pltpu.make_async_copy(hbf_hbm.at[srcs[r]], rows[p], gsem[p]).wait()
            _scale(p)
            pltpu.async_copy(srows[p], acc.at[dsts[r]], ssem[p], add=True)
            pltpu.async_copy(ws[p], accd.at[dsts[r]], ssem[p], add=True)

    def _six(kk, _):
        for off in range(6):
            _phase(6 * kk + off, off % 2, off % 3)
        return 0

    lax.fori_loop(0, (KMAX + 6) // 6, _six, 0)
    plsc.subcore_barrier()

    for cpy in range(RPT // C):
        r = sid * RPT + cpy * C
        pltpu.sync_copy(acc.at[pl.ds(r, C)], srows0)
        pltpu.sync_copy(srows0, num_hbm.at[cid, pl.ds(r, C)])
    pltpu.sync_copy(accd.at[pl.ds(sid * RPT, RPT)], zd_v)
    pltpu.sync_copy(zd_v, den_hbm.at[pl.ds(cid * NPAD + sid * RPT, RPT)])


_sc_edge = functools.partial(
    pl.kernel,
    out_type=[
        jax.ShapeDtypeStruct((NC, NPAD, D), jnp.float32),
        jax.ShapeDtypeStruct((NC * NPAD,), jnp.float32),
        jax.ShapeDtypeStruct((N, D), jnp.bfloat16),
    ],
    mesh=plsc.VectorSubcoreMesh(
        core_axis_name="c", subcore_axis_name="s",
        num_cores=NC, num_subcores=NS),
    compiler_params=pltpu.CompilerParams(use_tc_tiling_on_sc=False, needs_layout_passes=False),
    scratch_types=[
        pltpu.VMEM_SHARED((NPAD, D), jnp.float32),
        pltpu.VMEM_SHARED((NPAD,), jnp.float32),
        pltpu.VMEM((N,), jnp.int32),
        pltpu.VMEM((C,), jnp.int32),
        pltpu.VMEM((C,), jnp.int32),
        pltpu.VMEM((C,), jnp.int32),
        pltpu.VMEM((C,), jnp.int32),
        pltpu.VMEM((C,), jnp.int32),
        pltpu.VMEM((C,), jnp.int32),
        pltpu.VMEM((C,), jnp.float32),
        pltpu.VMEM((C,), jnp.float32),
        pltpu.VMEM((C, D), jnp.bfloat16),
        pltpu.VMEM((C, D), jnp.bfloat16),
        pltpu.VMEM((C, D), jnp.float32),
        pltpu.VMEM((C, D), jnp.float32),
        pltpu.VMEM((RPT,), jnp.float32),
        pltpu.SemaphoreType.DMA,
        pltpu.SemaphoreType.DMA,
        pltpu.SemaphoreType.DMA,
        pltpu.SemaphoreType.DMA,
        pltpu.SemaphoreType.DMA,
        pltpu.SemaphoreType.DMA,
        pltpu.SemaphoreType.DMA,
    ],
)(_sc_body)


def kernel(x, edge_index, W1, att_src1, att_dst1, b1, W2, att_src2, att_dst2, b2):
    src = edge_index[0].astype(jnp.int32)
    dst = edge_index[1].astype(jnp.int32)

    h1, pk1 = _tc_in(
        x, W1, att_src1.reshape(D, 1), att_dst1.reshape(D, 1))
    n1, d1, _hb1 = _sc_edge(h1, pk1.reshape(N), src, dst)
    h2, pk2 = _tc_mid(
        n1, d1.reshape(NC, NPAD, 1), b1.reshape(1, D), W2,
        att_src2.reshape(D, 1), att_dst2.reshape(D, 1))
    n2, d2, _hb2 = _sc_edge(h2, pk2.reshape(N), src, dst)
    return _tc_out(n2, d2.reshape(NC, NPAD, 1), b2.reshape(1, D))


# speedup vs baseline: 2.0650x; 2.0650x over previous
import functools

import jax
import jax.numpy as jnp
from jax import lax
from jax.experimental import pallas as pl
from jax.experimental.pallas import tpu as pltpu
from jax.experimental.pallas import tpu_sc as plsc

N = 10000
NPAD = 10240
E = 320000
D = 128
NC = 2
NS = 16
NW = NC * NS
C = 128
NCHUNK = E // C
KMAX = -(-NCHUNK // NW)
RPT = NPAD // NS
BN = 400
GRID = N // BN


def _pack_logits(a_s, a_d):
    su = lax.bitcast_convert_type(a_s.astype(jnp.bfloat16), jnp.uint16)
    du = lax.bitcast_convert_type(a_d.astype(jnp.bfloat16), jnp.uint16)
    return (du.astype(jnp.int32) << 16) | su.astype(jnp.int32)


def _tc_in_body(x_ref, w_ref, as_ref, ad_ref, h_ref, pk_ref):
    h = jnp.dot(x_ref[...], w_ref[...], preferred_element_type=jnp.float32)
    h_ref[...] = h
    a_s = jnp.dot(h, as_ref[...], preferred_element_type=jnp.float32)
    a_d = jnp.dot(h, ad_ref[...], preferred_element_type=jnp.float32)
    pk_ref[...] = _pack_logits(a_s, a_d)


def _tc_mid_body(np_ref, dp_ref, b1_ref, w_ref, as_ref, ad_ref,
                 h_ref, pk_ref):
    num = np_ref[0] + np_ref[1]
    den = dp_ref[0] + dp_ref[1]
    h1 = jnp.maximum(num / (den + 1e-16) + b1_ref[...], 0.0)
    h2 = jnp.dot(h1, w_ref[...], preferred_element_type=jnp.float32)
    h_ref[...] = h2
    a_s = jnp.dot(h2, as_ref[...], preferred_element_type=jnp.float32)
    a_d = jnp.dot(h2, ad_ref[...], preferred_element_type=jnp.float32)
    pk_ref[...] = _pack_logits(a_s, a_d)


def _tc_out_body(np_ref, dp_ref, b2_ref, out_ref):
    num = np_ref[0] + np_ref[1]
    den = dp_ref[0] + dp_ref[1]
    out_ref[...] = num / (den + 1e-16) + b2_ref[...]


_tc_in = pl.pallas_call(
    _tc_in_body,
    grid=(GRID,),
    in_specs=[
        pl.BlockSpec((BN, D), lambda i: (i, 0)),
        pl.BlockSpec((D, D), lambda i: (0, 0)),
        pl.BlockSpec((D, 1), lambda i: (0, 0)),
        pl.BlockSpec((D, 1), lambda i: (0, 0)),
    ],
    out_specs=[
        pl.BlockSpec((BN, D), lambda i: (i, 0)),
        pl.BlockSpec((BN, 1), lambda i: (i, 0)),
    ],
    out_shape=[
        jax.ShapeDtypeStruct((N, D), jnp.float32),
        jax.ShapeDtypeStruct((N, 1), jnp.int32),
    ],
)

_tc_mid = pl.pallas_call(
    _tc_mid_body,
    grid=(GRID,),
    in_specs=[
        pl.BlockSpec((2, BN, D), lambda i: (0, i, 0)),
        pl.BlockSpec((2, BN, 1), lambda i: (0, i, 0)),
        pl.BlockSpec((1, D), lambda i: (0, 0)),
        pl.BlockSpec((D, D), lambda i: (0, 0)),
        pl.BlockSpec((D, 1), lambda i: (0, 0)),
        pl.BlockSpec((D, 1), lambda i: (0, 0)),
    ],
    out_specs=[
        pl.BlockSpec((BN, D), lambda i: (i, 0)),
        pl.BlockSpec((BN, 1), lambda i: (i, 0)),
    ],
    out_shape=[
        jax.ShapeDtypeStruct((N, D), jnp.float32),
        jax.ShapeDtypeStruct((N, 1), jnp.int32),
    ],
)

_tc_out = pl.pallas_call(
    _tc_out_body,
    grid=(GRID,),
    in_specs=[
        pl.BlockSpec((2, BN, D), lambda i: (0, i, 0)),
        pl.BlockSpec((2, BN, 1), lambda i: (0, i, 0)),
        pl.BlockSpec((1, D), lambda i: (0, 0)),
    ],
    out_specs=pl.BlockSpec((BN, D), lambda i: (i, 0)),
    out_shape=jax.ShapeDtypeStruct((N, D), jnp.float32),
)


def _sc_body(h_hbm, pk_hbm, src_hbm, dst_hbm,
             num_hbm, den_hbm,
             acc, accd,
             pk_t,
             src0, src1, src2, dst0, dst1, dst2, w0, w1, rows0, rows1, zd_v,
             g0, g1, s0, s1, i0, i1, i2):
    cid = lax.axis_index("c")
    sid = lax.axis_index("s")
    wkr = sid * NC + cid

    srcs = (src0, src1, src2)
    dsts = (dst0, dst1, dst2)
    ws = (w0, w1)
    rows = (rows0, rows1)
    gsem = (g0, g1)
    ssem = (s0, s1)
    isem = (i0, i1, i2)

    pltpu.sync_copy(pk_hbm, pk_t)

    zeros16 = jnp.zeros((16,), jnp.float32)

    def _zero_row(i, _):
        for d in range(D // 16):
            rows0[i, pl.ds(16 * d, 16)] = zeros16
        return 0

    lax.fori_loop(0, C, _zero_row, 0)

    def _zero_zd(i, _):
        zd_v[pl.ds(16 * i, 16)] = zeros16
        return 0

    lax.fori_loop(0, RPT // 16, _zero_zd, 0)

    for cpy in range(RPT // C):
        pltpu.sync_copy(rows0, acc.at[pl.ds(sid * RPT + cpy * C, C)])
    pltpu.sync_copy(zd_v, accd.at[pl.ds(sid * RPT, RPT)])
    plsc.subcore_barrier()

    def _compute_w(p, r):
        himask = jnp.full((16,), -65536, jnp.int32)
        for j in range(C // 16):
            si = srcs[r][pl.ds(16 * j, 16)]
            di = dsts[r][pl.ds(16 * j, 16)]
            vs = plsc.load_gather(pk_t, [si])
            vd = plsc.load_gather(pk_t, [di])
            a_s = plsc.bitcast(vs << 16, jnp.float32)
            a_d = plsc.bitcast(vd & himask, jnp.float32)
            z = a_s + a_d
            e = jnp.where(z >= 0.0, z, z * 0.2)
            ws[p][pl.ds(16 * j, 16)] = jnp.exp(e)

    def _scale(p):
        def body(j, _):
            wv = ws[p][pl.ds(16 * j, 16)]
            for e in range(16):
                wb = jnp.full((16,), wv[e], jnp.float32)
                row = 16 * j + e
                for d in range(D // 16):
                    sl = pl.ds(16 * d, 16)
                    rows[p][row, sl] = rows[p][row, sl] * wb
            return 0

        lax.fori_loop(0, C // 16, body, 0)

    def _fetch_idx(k, r):
        base = (k * NW + wkr) * C
        pltpu.async_copy(src_hbm.at[pl.ds(base, C)], srcs[r], isem[r])
        pltpu.async_copy(dst_hbm.at[pl.ds(base, C)], dsts[r], isem[r])

    def _drain_idx(r):
        pltpu.make_async_copy(src_hbm.at[pl.ds(0, C)], srcs[r], isem[r]).wait()
        pltpu.make_async_copy(dst_hbm.at[pl.ds(0, C)], dsts[r], isem[r]).wait()

    _fetch_idx(0, 0)
    _fetch_idx(1, 1)
    _drain_idx(0)
    pltpu.async_copy(h_hbm.at[srcs[0]], rows[0], gsem[0])

    def _phase(k, p, r):
        q = 1 - p
        r1 = (r + 1) % 3
        r2 = (r + 2) % 3
        ck = k * NW + wkr
        valid = ck < NCHUNK
        prev_valid = jnp.logical_and(k >= 1, (k - 1) * NW + wkr < NCHUNK)
        next_valid = (k + 1) * NW + wkr < NCHUNK
        next2_valid = (k + 2) * NW + wkr < NCHUNK

        @pl.when(prev_valid)
        def _():
            pltpu.make_async_copy(rows[q], acc.at[dsts[r2]], ssem[q]).wait()
            pltpu.make_async_copy(ws[q], accd.at[dsts[r2]], ssem[q]).wait()

        @pl.when(next2_valid)
        def _():
            _fetch_idx(k + 2, r2)

        @pl.when(next_valid)
        def _():
            _drain_idx(r1)
            pltpu.async_copy(h_hbm.at[srcs[r1]], rows[q], gsem[q])

        @pl.when(valid)
        def _():
            _compute_w(p, r)
            pltpu.make_async_copy(h_hbm.at[srcs[r]], rows[p], gsem[p]).wait()
            _scale(p)
            pltpu.async_copy(rows[p], acc.at[dsts[r]], ssem[p], add=True)
            pltpu.async_copy(ws[p], accd.at[dsts[r]], ssem[p], add=True)

    def _six(kk, _):
        for off in range(6):
            _phase(6 * kk + off, off % 2, off % 3)
        return 0

    lax.fori_loop(0, (KMAX + 6) // 6, _six, 0)
    plsc.subcore_barrier()

    for cpy in range(RPT // C):
        r = sid * RPT + cpy * C
        pltpu.sync_copy(acc.at[pl.ds(r, C)], rows0)
        pltpu.sync_copy(rows0, num_hbm.at[cid, pl.ds(r, C)])
    pltpu.sync_copy(accd.at[pl.ds(sid * RPT, RPT)], zd_v)
    pltpu.sync_copy(zd_v, den_hbm.at[pl.ds(cid * NPAD + sid * RPT, RPT)])


_sc_edge = functools.partial(
    pl.kernel,
    out_type=[
        jax.ShapeDtypeStruct((NC, NPAD, D), jnp.float32),
        jax.ShapeDtypeStruct((NC * NPAD,), jnp.float32),
    ],
    mesh=plsc.VectorSubcoreMesh(
        core_axis_name="c", subcore_axis_name="s",
        num_cores=NC, num_subcores=NS),
    compiler_params=pltpu.CompilerParams(use_tc_tiling_on_sc=False, needs_layout_passes=False),
    scratch_types=[
        pltpu.VMEM_SHARED((NPAD, D), jnp.float32),
        pltpu.VMEM_SHARED((NPAD,), jnp.float32),
        pltpu.VMEM((N,), jnp.int32),
        pltpu.VMEM((C,), jnp.int32),
        pltpu.VMEM((C,), jnp.int32),
        pltpu.VMEM((C,), jnp.int32),
        pltpu.VMEM((C,), jnp.int32),
        pltpu.VMEM((C,), jnp.int32),
        pltpu.VMEM((C,), jnp.int32),
        pltpu.VMEM((C,), jnp.float32),
        pltpu.VMEM((C,), jnp.float32),
        pltpu.VMEM((C, D), jnp.float32),
        pltpu.VMEM((C, D), jnp.float32),
        pltpu.VMEM((RPT,), jnp.float32),
        pltpu.SemaphoreType.DMA,
        pltpu.SemaphoreType.DMA,
        pltpu.SemaphoreType.DMA,
        pltpu.SemaphoreType.DMA,
        pltpu.SemaphoreType.DMA,
        pltpu.SemaphoreType.DMA,
        pltpu.SemaphoreType.DMA,
    ],
)(_sc_body)


def kernel(x, edge_index, W1, att_src1, att_dst1, b1, W2, att_src2, att_dst2, b2):
    src = edge_index[0].astype(jnp.int32)
    dst = edge_index[1].astype(jnp.int32)

    h1, pk1 = _tc_in(
        x, W1, att_src1.reshape(D, 1), att_dst1.reshape(D, 1))
    n1, d1 = _sc_edge(h1, pk1.reshape(N), src, dst)
    h2, pk2 = _tc_mid(
        n1, d1.reshape(NC, NPAD, 1), b1.reshape(1, D), W2,
        att_src2.reshape(D, 1), att_dst2.reshape(D, 1))
    n2, d2 = _sc_edge(h2, pk2.reshape(N), src, dst)
    return _tc_out(n2, d2.reshape(NC, NPAD, 1), b2.reshape(1, D))


# speedup vs baseline: 2.0661x; 1.0005x over previous
import functools

import jax
import jax.numpy as jnp
from jax import lax
from jax.experimental import pallas as pl
from jax.experimental.pallas import tpu as pltpu
from jax.experimental.pallas import tpu_sc as plsc

N = 10000
NPAD = 10240
E = 320000
D = 128
NC = 2
NS = 16
NW = NC * NS
C = 128
NCHUNK = E // C
KMAX = -(-NCHUNK // NW)
RPT = NPAD // NS
BN = 400
GRID = N // BN


def _pack_logits(a_s, a_d):
    su = lax.bitcast_convert_type(a_s.astype(jnp.bfloat16), jnp.uint16)
    du = lax.bitcast_convert_type(a_d.astype(jnp.bfloat16), jnp.uint16)
    return (du.astype(jnp.int32) << 16) | su.astype(jnp.int32)


def _tc_in_body(x_ref, w_ref, as_ref, ad_ref, h_ref, pk_ref):
    h = jnp.dot(x_ref[...], w_ref[...], preferred_element_type=jnp.float32)
    h_ref[...] = h
    a_s = jnp.dot(h, as_ref[...], preferred_element_type=jnp.float32)
    a_d = jnp.dot(h, ad_ref[...], preferred_element_type=jnp.float32)
    pk_ref[...] = _pack_logits(a_s, a_d)


def _tc_mid_body(np_ref, dp_ref, b1_ref, w_ref, as_ref, ad_ref,
                 h_ref, pk_ref):
    num = np_ref[0] + np_ref[1]
    den = dp_ref[0] + dp_ref[1]
    h1 = jnp.maximum(num / (den + 1e-16) + b1_ref[...], 0.0)
    h2 = jnp.dot(h1, w_ref[...], preferred_element_type=jnp.float32)
    h_ref[...] = h2
    a_s = jnp.dot(h2, as_ref[...], preferred_element_type=jnp.float32)
    a_d = jnp.dot(h2, ad_ref[...], preferred_element_type=jnp.float32)
    pk_ref[...] = _pack_logits(a_s, a_d)


def _tc_out_body(np_ref, dp_ref, b2_ref, out_ref):
    num = np_ref[0] + np_ref[1]
    den = dp_ref[0] + dp_ref[1]
    out_ref[...] = num / (den + 1e-16) + b2_ref[...]


_tc_in = pl.pallas_call(
    _tc_in_body,
    grid=(GRID,),
    in_specs=[
        pl.BlockSpec((BN, D), lambda i: (i, 0)),
        pl.BlockSpec((D, D), lambda i: (0, 0)),
        pl.BlockSpec((D, 1), lambda i: (0, 0)),
        pl.BlockSpec((D, 1), lambda i: (0, 0)),
    ],
    out_specs=[
        pl.BlockSpec((BN, D), lambda i: (i, 0)),
        pl.BlockSpec((BN, 1), lambda i: (i, 0)),
    ],
    out_shape=[
        jax.ShapeDtypeStruct((N, D), jnp.float32),
        jax.ShapeDtypeStruct((N, 1), jnp.int32),
    ],
)

_tc_mid = pl.pallas_call(
    _tc_mid_body,
    grid=(GRID,),
    in_specs=[
        pl.BlockSpec((2, BN, D), lambda i: (0, i, 0)),
        pl.BlockSpec((2, BN, 1), lambda i: (0, i, 0)),
        pl.BlockSpec((1, D), lambda i: (0, 0)),
        pl.BlockSpec((D, D), lambda i: (0, 0)),
        pl.BlockSpec((D, 1), lambda i: (0, 0)),
        pl.BlockSpec((D, 1), lambda i: (0, 0)),
    ],
    out_specs=[
        pl.BlockSpec((BN, D), lambda i: (i, 0)),
        pl.BlockSpec((BN, 1), lambda i: (i, 0)),
    ],
    out_shape=[
        jax.ShapeDtypeStruct((N, D), jnp.float32),
        jax.ShapeDtypeStruct((N, 1), jnp.int32),
    ],
)

_tc_out = pl.pallas_call(
    _tc_out_body,
    grid=(GRID,),
    in_specs=[
        pl.BlockSpec((2, BN, D), lambda i: (0, i, 0)),
        pl.BlockSpec((2, BN, 1), lambda i: (0, i, 0)),
        pl.BlockSpec((1, D), lambda i: (0, 0)),
    ],
    out_specs=pl.BlockSpec((BN, D), lambda i: (i, 0)),
    out_shape=jax.ShapeDtypeStruct((N, D), jnp.float32),
)


def _sc_body(h_hbm, pk_hbm, src_hbm, dst_hbm,
             num_hbm, den_hbm,
             acc, accd,
             pk_t,
             src0, src1, src2, dst0, dst1, dst2, w0, w1, rows0, rows1, zd_v,
             g0, g1, s0, s1, i0, i1, i2):
    cid = lax.axis_index("c")
    sid = lax.axis_index("s")
    wkr = sid * NC + cid

    srcs = (src0, src1, src2)
    dsts = (dst0, dst1, dst2)
    ws = (w0, w1)
    rows = (rows0, rows1)
    gsem = (g0, g1)
    ssem = (s0, s1)
    isem = (i0, i1, i2)

    pltpu.sync_copy(pk_hbm, pk_t)

    zeros16 = jnp.zeros((16,), jnp.float32)

    def _zero_row(i, _):
        for d in range(D // 16):
            rows0[i, pl.ds(16 * d, 16)] = zeros16
        return 0

    lax.fori_loop(0, C, _zero_row, 0)

    def _zero_zd(i, _):
        zd_v[pl.ds(16 * i, 16)] = zeros16
        return 0

    lax.fori_loop(0, RPT // 16, _zero_zd, 0)

    for cpy in range(RPT // C):
        pltpu.sync_copy(rows0, acc.at[pl.ds(sid * RPT + cpy * C, C)])
    pltpu.sync_copy(zd_v, accd.at[pl.ds(sid * RPT, RPT)])
    plsc.subcore_barrier()

    def _compute_w(p, r):
        himask = jnp.full((16,), -65536, jnp.int32)
        for j in range(C // 16):
            si = srcs[r][pl.ds(16 * j, 16)]
            di = dsts[r][pl.ds(16 * j, 16)]
            vs = plsc.load_gather(pk_t, [si])
            vd = plsc.load_gather(pk_t, [di])
            a_s = plsc.bitcast(vs << 16, jnp.float32)
            a_d = plsc.bitcast(vd & himask, jnp.float32)
            z = a_s + a_d
            e = jnp.where(z >= 0.0, z, z * 0.2)
            ws[p][pl.ds(16 * j, 16)] = jnp.exp(e)

    def _scale(p):
        def body(j, _):
            wv = ws[p][pl.ds(16 * j, 16)]
            for e in range(16):
                wb = jnp.full((16,), wv[e], jnp.float32)
                row = 16 * j + e
                for d in range(D // 16):
                    sl = pl.ds(16 * d, 16)
                    rows[p][row, sl] = rows[p][row, sl] * wb
            return 0

        lax.fori_loop(0, C // 16, body, 0)

    def _fetch_idx(k, r):
        base = (k * NW + wkr) * C
        pltpu.async_copy(src_hbm.at[pl.ds(base, C)], srcs[r], isem[r])
        pltpu.async_copy(dst_hbm.at[pl.ds(base, C)], dsts[r], isem[r])

    def _gather(r, p):
        pltpu.async_copy(h_hbm.at[srcs[r].at[pl.ds(0, C // 2)]],
                         rows[p].at[pl.ds(0, C // 2)], gsem[p])
        pltpu.async_copy(h_hbm.at[srcs[r].at[pl.ds(C // 2, C // 2)]],
                         rows[p].at[pl.ds(C // 2, C // 2)], gsem[p])

    def _drain_gather(r, p):
        pltpu.make_async_copy(h_hbm.at[srcs[r].at[pl.ds(0, C // 2)]],
                              rows[p].at[pl.ds(0, C // 2)], gsem[p]).wait()
        pltpu.make_async_copy(h_hbm.at[srcs[r].at[pl.ds(C // 2, C // 2)]],
                              rows[p].at[pl.ds(C // 2, C // 2)], gsem[p]).wait()

    def _drain_idx(r):
        pltpu.make_async_copy(src_hbm.at[pl.ds(0, C)], srcs[r], isem[r]).wait()
        pltpu.make_async_copy(dst_hbm.at[pl.ds(0, C)], dsts[r], isem[r]).wait()

    _fetch_idx(0, 0)
    _fetch_idx(1, 1)
    _drain_idx(0)
    _gather(0, 0)

    def _phase(k, p, r):
        q = 1 - p
        r1 = (r + 1) % 3
        r2 = (r + 2) % 3
        ck = k * NW + wkr
        valid = ck < NCHUNK
        prev_valid = jnp.logical_and(k >= 1, (k - 1) * NW + wkr < NCHUNK)
        next_valid = (k + 1) * NW + wkr < NCHUNK
        next2_valid = (k + 2) * NW + wkr < NCHUNK

        @pl.when(prev_valid)
        def _():
            pltpu.make_async_copy(rows[q], acc.at[dsts[r2]], ssem[q]).wait()
            pltpu.make_async_copy(ws[q], accd.at[dsts[r2]], ssem[q]).wait()

        @pl.when(next2_valid)
        def _():
            _fetch_idx(k + 2, r2)

        @pl.when(next_valid)
        def _():
            _drain_idx(r1)
            _gather(r1, q)

        @pl.when(valid)
        def _():
            _compute_w(p, r)
            _drain_gather(r, p)
            _scale(p)
            pltpu.async_copy(rows[p], acc.at[dsts[r]], ssem[p], add=True)
            pltpu.async_copy(ws[p], accd.at[dsts[r]], ssem[p], add=True)

    def _six(kk, _):
        for off in range(6):
            _phase(6 * kk + off, off % 2, off % 3)
        return 0

    lax.fori_loop(0, (KMAX + 6) // 6, _six, 0)
    plsc.subcore_barrier()

    for cpy in range(RPT // C):
        r = sid * RPT + cpy * C
        pltpu.sync_copy(acc.at[pl.ds(r, C)], rows0)
        pltpu.sync_copy(rows0, num_hbm.at[cid, pl.ds(r, C)])
    pltpu.sync_copy(accd.at[pl.ds(sid * RPT, RPT)], zd_v)
    pltpu.sync_copy(zd_v, den_hbm.at[pl.ds(cid * NPAD + sid * RPT, RPT)])


_sc_edge = functools.partial(
    pl.kernel,
    out_type=[
        jax.ShapeDtypeStruct((NC, NPAD, D), jnp.float32),
        jax.ShapeDtypeStruct((NC * NPAD,), jnp.float32),
    ],
    mesh=plsc.VectorSubcoreMesh(
        core_axis_name="c", subcore_axis_name="s",
        num_cores=NC, num_subcores=NS),
    compiler_params=pltpu.CompilerParams(use_tc_tiling_on_sc=False, needs_layout_passes=False),
    scratch_types=[
        pltpu.VMEM_SHARED((NPAD, D), jnp.float32),
        pltpu.VMEM_SHARED((NPAD,), jnp.float32),
        pltpu.VMEM((N,), jnp.int32),
        pltpu.VMEM((C,), jnp.int32),
        pltpu.VMEM((C,), jnp.int32),
        pltpu.VMEM((C,), jnp.int32),
        pltpu.VMEM((C,), jnp.int32),
        pltpu.VMEM((C,), jnp.int32),
        pltpu.VMEM((C,), jnp.int32),
        pltpu.VMEM((C,), jnp.float32),
        pltpu.VMEM((C,), jnp.float32),
        pltpu.VMEM((C, D), jnp.float32),
        pltpu.VMEM((C, D), jnp.float32),
        pltpu.VMEM((RPT,), jnp.float32),
        pltpu.SemaphoreType.DMA,
        pltpu.SemaphoreType.DMA,
        pltpu.SemaphoreType.DMA,
        pltpu.SemaphoreType.DMA,
        pltpu.SemaphoreType.DMA,
        pltpu.SemaphoreType.DMA,
        pltpu.SemaphoreType.DMA,
    ],
)(_sc_body)


def kernel(x, edge_index, W1, att_src1, att_dst1, b1, W2, att_src2, att_dst2, b2):
    src = edge_index[0].astype(jnp.int32)
    dst = edge_index[1].astype(jnp.int32)

    h1, pk1 = _tc_in(
        x, W1, att_src1.reshape(D, 1), att_dst1.reshape(D, 1))
    n1, d1 = _sc_edge(h1, pk1.reshape(N), src, dst)
    h2, pk2 = _tc_mid(
        n1, d1.reshape(NC, NPAD, 1), b1.reshape(1, D), W2,
        att_src2.reshape(D, 1), att_dst2.reshape(D, 1))
    n2, d2 = _sc_edge(h2, pk2.reshape(N), src, dst)
    return _tc_out(n2, d2.reshape(NC, NPAD, 1), b2.reshape(1, D))
